# Initial kernel scaffold; baseline (speedup 1.0000x reference)
import jax
import jax.numpy as jnp
from jax.experimental import pallas as pl


def kernel(x_seq, ei_seq, W1, b1, W2, b2, W_ih, W_hh, b_ih, b_hh, Wh, bh):
    raise NotImplementedError("write your pallas kernel here")


# speedup vs baseline: 9.9221x; 9.9221x over previous
import functools

import jax
import jax.numpy as jnp
from jax import lax
from jax.experimental import pallas as pl
from jax.experimental.pallas import tpu as pltpu
from jax.experimental.pallas import tpu_sc as plsc

T = 8
N = 10000
N_PAD = 10240
E = 320000
D = 128
H = 128

NC = 2
NS = 16
NW = NC * NS
EPW = E // NW
CH = 80
NCH = EPW // CH
RPS = N_PAD // NS
ZR = 64

BN = 1024
NB = N_PAD // BN


def _mesh():
    return plsc.VectorSubcoreMesh(core_axis_name="c", subcore_axis_name="s")


@functools.partial(
    pl.kernel,
    mesh=_mesh(),
    out_type=jax.ShapeDtypeStruct((T * NC * N_PAD,), jnp.float32),
    scratch_types=[
        pltpu.VMEM((CH,), jnp.int32),
        pltpu.VMEM((CH,), jnp.float32),
        pltpu.VMEM((RPS,), jnp.float32),
        pltpu.VMEM_SHARED((N_PAD,), jnp.float32),
    ],
)
def _sc_deg(dst_hbm, out_hbm, didx, ones_v, zv, acc):
    cid = lax.axis_index("c")
    sid = lax.axis_index("s")
    base = (cid * NS + sid) * EPW

    def fill(i, _):
        ones_v[pl.ds(i * 16, 16)] = jnp.full((16,), 1.0, jnp.float32)
        return 0
    lax.fori_loop(0, CH // 16, fill, 0)

    def zfill(i, _):
        zv[pl.ds(i * 16, 16)] = jnp.zeros((16,), jnp.float32)
        return 0
    lax.fori_loop(0, RPS // 16, zfill, 0)

    for t in range(T):
        pltpu.sync_copy(zv, acc.at[pl.ds(sid * RPS, RPS)])
        plsc.subcore_barrier()

        def body(j, _):
            off = t * E + base + j * CH
            pltpu.sync_copy(dst_hbm.at[pl.ds(off, CH)], didx)
            pltpu.sync_copy(ones_v, acc.at[didx], add=True)
            return 0
        lax.fori_loop(0, NCH, body, 0)
        plsc.subcore_barrier()

        pltpu.sync_copy(acc.at[pl.ds(sid * RPS, RPS)],
                        out_hbm.at[pl.ds((t * NC + cid) * N_PAD + sid * RPS, RPS)])
        plsc.subcore_barrier()


@functools.partial(
    pl.kernel,
    mesh=_mesh(),
    out_type=jax.ShapeDtypeStruct((T * NC * N_PAD, D), jnp.float32),
    scratch_types=[
        pltpu.VMEM((CH,), jnp.int32),
        pltpu.VMEM((CH,), jnp.int32),
        pltpu.VMEM((CH, D), jnp.float32),
        pltpu.VMEM((ZR, D), jnp.float32),
        pltpu.VMEM_SHARED((N_PAD, D), jnp.float32),
        pltpu.SemaphoreType.DMA,
    ],
)
def _sc_edge(hn_hbm, src_hbm, dst_hbm, out_hbm, sidx, didx, rows, zbuf, acc, sem):
    cid = lax.axis_index("c")
    sid = lax.axis_index("s")
    base = (cid * NS + sid) * EPW

    def zfill(i, _):
        r = i // (D // 16)
        c = (i % (D // 16)) * 16
        zbuf[r, pl.ds(c, 16)] = jnp.zeros((16,), jnp.float32)
        return 0
    lax.fori_loop(0, ZR * (D // 16), zfill, 0)

    for t in range(T):
        def zero(i, _):
            pltpu.sync_copy(zbuf, acc.at[pl.ds(sid * RPS + i * ZR, ZR)])
            return 0
        lax.fori_loop(0, RPS // ZR, zero, 0)
        plsc.subcore_barrier()

        def body(j, _):
            off = t * E + base + j * CH
            pltpu.sync_copy(src_hbm.at[pl.ds(off, CH)], sidx)
            pltpu.sync_copy(dst_hbm.at[pl.ds(off, CH)], didx)
            pltpu.async_copy(hn_hbm.at[sidx], rows, sem).wait()
            pltpu.sync_copy(rows, acc.at[didx], add=True)
            return 0
        lax.fori_loop(0, NCH, body, 0)
        plsc.subcore_barrier()

        pltpu.sync_copy(acc.at[pl.ds(sid * RPS, RPS)],
                        out_hbm.at[pl.ds((t * NC + cid) * N_PAD + sid * RPS, RPS)])
        plsc.subcore_barrier()


def _m1_body(x_ref, w_ref, deg_ref, out_ref):
    dsum = deg_ref[0, 0] + deg_ref[0, 1]
    dinv = lax.rsqrt(dsum + 1.0)
    h = jnp.dot(x_ref[0], w_ref[...], preferred_element_type=jnp.float32)
    out_ref[...] = h * dinv


def _tc_m1(x_pad, W1, degcol):
    return pl.pallas_call(
        _m1_body,
        grid=(T, NB),
        in_specs=[
            pl.BlockSpec((1, BN, D), lambda t, n: (t, n, 0)),
            pl.BlockSpec((D, H), lambda t, n: (0, 0)),
            pl.BlockSpec((1, NC, BN, 1), lambda t, n: (t, 0, n, 0)),
        ],
        out_specs=pl.BlockSpec((BN, H), lambda t, n: (t * NB + n, 0)),
        out_shape=jax.ShapeDtypeStruct((T * N_PAD, H), jnp.float32),
    )(x_pad, W1, degcol)


def _m2_body(acc_ref, hn_ref, deg_ref, b_ref, w_ref, out_ref):
    dsum = deg_ref[0, 0] + deg_ref[0, 1]
    dinv = lax.rsqrt(dsum + 1.0)
    s = acc_ref[0, 0] + acc_ref[0, 1] + hn_ref[...]
    h1 = jnp.maximum(s * dinv + b_ref[...], 0.0)
    out_ref[...] = jnp.dot(h1, w_ref[...], preferred_element_type=jnp.float32) * dinv


def _tc_m2(acc1, hn1, degcol, b1r, W2):
    return pl.pallas_call(
        _m2_body,
        grid=(T, NB),
        in_specs=[
            pl.BlockSpec((1, NC, BN, D), lambda t, n: (t, 0, n, 0)),
            pl.BlockSpec((BN, D), lambda t, n: (t * NB + n, 0)),
            pl.BlockSpec((1, NC, BN, 1), lambda t, n: (t, 0, n, 0)),
            pl.BlockSpec((1, H), lambda t, n: (0, 0)),
            pl.BlockSpec((H, H), lambda t, n: (0, 0)),
        ],
        out_specs=pl.BlockSpec((BN, H), lambda t, n: (t * NB + n, 0)),
        out_shape=jax.ShapeDtypeStruct((T * N_PAD, H), jnp.float32),
    )(acc1, hn1, degcol, b1r, W2)


def _f_body(acc_ref, hn_ref, deg_ref, b_ref, out_ref):
    n = pl.program_id(1)
    dsum = deg_ref[0, 0] + deg_ref[0, 1]
    dinv = lax.rsqrt(dsum + 1.0)
    s = acc_ref[0, 0] + acc_ref[0, 1] + hn_ref[...]
    h2 = jnp.maximum(s * dinv + b_ref[...], 0.0)
    rowid = lax.broadcasted_iota(jnp.int32, (BN, 1), 0) + n * BN
    h2 = jnp.where(rowid < N, h2, 0.0)

    @pl.when(n == 0)
    def _():
        out_ref[...] = jnp.zeros_like(out_ref)

    out_ref[...] += jnp.sum(h2, axis=0, keepdims=True)[None]


def _tc_f(acc2, hn2, degcol, b2r):
    return pl.pallas_call(
        _f_body,
        grid=(T, NB),
        in_specs=[
            pl.BlockSpec((1, NC, BN, D), lambda t, n: (t, 0, n, 0)),
            pl.BlockSpec((BN, D), lambda t, n: (t * NB + n, 0)),
            pl.BlockSpec((1, NC, BN, 1), lambda t, n: (t, 0, n, 0)),
            pl.BlockSpec((1, H), lambda t, n: (0, 0)),
        ],
        out_specs=pl.BlockSpec((1, 1, H), lambda t, n: (t, 0, 0)),
        out_shape=jax.ShapeDtypeStruct((T, 1, H), jnp.float32),
    )(acc2, hn2, degcol, b2r)


def _gru_body(g_ref, wih_ref, whh_ref, bih_ref, bhh_ref, wh_ref, bh_ref, out_ref):
    g = g_ref[...] * (1.0 / N)
    wih = wih_ref[...]
    whh = whh_ref[...]
    bih = bih_ref[...]
    bhh = bhh_ref[...]
    dn = (((1,), (1,)), ((), ()))
    h = jnp.zeros((1, H), jnp.float32)
    for t in range(T):
        xt = g[t:t + 1, :]
        gi = lax.dot_general(xt, wih, dn, preferred_element_type=jnp.float32) + bih
        gh = lax.dot_general(h, whh, dn, preferred_element_type=jnp.float32) + bhh
        r = jax.nn.sigmoid(gi[:, :H] + gh[:, :H])
        z = jax.nn.sigmoid(gi[:, H:2 * H] + gh[:, H:2 * H])
        n_ = jnp.tanh(gi[:, 2 * H:] + r * gh[:, 2 * H:])
        h = (1.0 - z) * n_ + z * h
    out_ref[...] = lax.dot_general(h, wh_ref[...], dn,
                                   preferred_element_type=jnp.float32) + bh_ref[...]


def _tc_gru(g, W_ih, W_hh, b_ihr, b_hhr, Wh, bhr):
    return pl.pallas_call(
        _gru_body,
        out_shape=jax.ShapeDtypeStruct((1, D), jnp.float32),
    )(g, W_ih, W_hh, b_ihr, b_hhr, Wh, bhr)


def kernel(x_seq, ei_seq, W1, b1, W2, b2, W_ih, W_hh, b_ih, b_hh, Wh, bh):
    src = ei_seq[:, 0, :]
    dst = ei_seq[:, 1, :].reshape(T * E)
    src_flat = (src + (jnp.arange(T, dtype=jnp.int32) * N_PAD)[:, None]).reshape(T * E)
    x_pad = jnp.concatenate(
        [x_seq, jnp.zeros((T, N_PAD - N, D), jnp.float32)], axis=1)

    deg2 = _sc_deg(dst)
    degcol = deg2.reshape(T, NC, N_PAD, 1)
    hn1 = _tc_m1(x_pad, W1, degcol)
    acc1 = _sc_edge(hn1, src_flat, dst).reshape(T, NC, N_PAD, H)
    hn2 = _tc_m2(acc1, hn1, degcol, b1.reshape(1, H), W2)
    acc2 = _sc_edge(hn2, src_flat, dst).reshape(T, NC, N_PAD, H)
    gsum = _tc_f(acc2, hn2, degcol, b2.reshape(1, H))
    out = _tc_gru(gsum.reshape(T, H), W_ih, W_hh,
                  b_ih.reshape(1, 3 * H), b_hh.reshape(1, 3 * H),
                  Wh, bh.reshape(1, D))
    return out.reshape(D)


# speedup vs baseline: 18.1971x; 1.8340x over previous
import functools

import jax
import jax.numpy as jnp
from jax import lax
from jax.experimental import pallas as pl
from jax.experimental.pallas import tpu as pltpu
from jax.experimental.pallas import tpu_sc as plsc

T = 8
N = 10000
N_PAD = 10240
E = 320000
D = 128
H = 128

NC = 2
NS = 16
NW = NC * NS
EPW = E // NW
CH = 40
NCH = EPW // CH
NSL = 4
NGI = (NCH + NSL - 1) // NSL
RPS = N_PAD // NS
ZR = 32

BN = 1024
NB = N_PAD // BN


def _mesh():
    return plsc.VectorSubcoreMesh(core_axis_name="c", subcore_axis_name="s")


DCH = 80
DNCH = EPW // DCH
DSL = 2
DNG = (DNCH + DSL - 1) // DSL


@functools.partial(
    pl.kernel,
    mesh=_mesh(),
    out_type=jax.ShapeDtypeStruct((T * NC * N_PAD,), jnp.float32),
    scratch_types=[
        pltpu.VMEM((DCH,), jnp.int32),
        pltpu.VMEM((DCH,), jnp.int32),
        pltpu.VMEM((DCH,), jnp.float32),
        pltpu.VMEM((RPS,), jnp.float32),
        pltpu.VMEM_SHARED((N_PAD,), jnp.float32),
        pltpu.SemaphoreType.DMA,
        pltpu.SemaphoreType.DMA,
    ],
)
def _sc_deg(dst_hbm, out_hbm, didx0, didx1, ones_v, zv, acc, sem0, sem1):
    didx = (didx0, didx1)
    dsem = (sem0, sem1)
    cid = lax.axis_index("c")
    sid = lax.axis_index("s")
    wid = cid * NS + sid

    def fill(i, _):
        ones_v[pl.ds(i * 16, 16)] = jnp.full((16,), 1.0, jnp.float32)
        return 0
    lax.fori_loop(0, DCH // 16, fill, 0)

    def zfill(i, _):
        zv[pl.ds(i * 16, 16)] = jnp.zeros((16,), jnp.float32)
        return 0
    lax.fori_loop(0, RPS // 16, zfill, 0)

    for t in range(T):
        ebase = t * E + wid * EPW
        pltpu.sync_copy(zv, acc.at[pl.ds(sid * RPS, RPS)])
        for q in range(DSL):
            pltpu.sync_copy(dst_hbm.at[pl.ds(ebase + q * DCH, DCH)], didx[q])
        plsc.subcore_barrier()

        def grp(g, _):
            for q in range(DSL):
                j = g * DSL + q

                @pl.when(j < DNCH)
                def _():
                    pltpu.sync_copy(ones_v, acc.at[didx[q]], add=True)

                    @pl.when(j + DSL < DNCH)
                    def _():
                        pltpu.make_async_copy(
                            dst_hbm.at[pl.ds(ebase + (j + DSL) * DCH, DCH)],
                            didx[q], dsem[q]).start()

            for q in range(DSL):
                j2 = (g + 1) * DSL + q

                @pl.when(j2 < DNCH)
                def _():
                    pltpu.make_async_copy(
                        dst_hbm.at[pl.ds(ebase + j2 * DCH, DCH)],
                        didx[q], dsem[q]).wait()
            return 0
        lax.fori_loop(0, DNG, grp, 0)
        plsc.subcore_barrier()

        pltpu.sync_copy(acc.at[pl.ds(sid * RPS, RPS)],
                        out_hbm.at[pl.ds((t * NC + cid) * N_PAD + sid * RPS, RPS)])
        plsc.subcore_barrier()


@functools.partial(
    pl.kernel,
    mesh=_mesh(),
    out_type=jax.ShapeDtypeStruct((T * NC * N_PAD, D), jnp.float32),
    scratch_types=(
        [pltpu.VMEM((CH,), jnp.int32) for _ in range(NSL)]
        + [pltpu.VMEM((CH,), jnp.int32) for _ in range(NSL)]
        + [pltpu.VMEM((CH, D), jnp.float32) for _ in range(NSL)]
        + [pltpu.VMEM((ZR, D), jnp.float32)]
        + [pltpu.VMEM_SHARED((N_PAD, D), jnp.float32)]
        + [pltpu.SemaphoreType.DMA] * (3 * NSL)
    ),
)
def _sc_edge(hn_hbm, src_hbm, dst_hbm, out_hbm, *refs):
    sidx = refs[0:NSL]
    didx = refs[NSL:2 * NSL]
    rows = refs[2 * NSL:3 * NSL]
    zbuf = refs[3 * NSL]
    acc = refs[3 * NSL + 1]
    ssem = refs[3 * NSL + 2:3 * NSL + 2 + NSL]
    dsem = refs[3 * NSL + 2 + NSL:3 * NSL + 2 + 2 * NSL]
    gsem = refs[3 * NSL + 2 + 2 * NSL:3 * NSL + 2 + 3 * NSL]

    cid = lax.axis_index("c")
    sid = lax.axis_index("s")
    wid = cid * NS + sid

    def zfill(i, _):
        r = i // (D // 16)
        c = (i % (D // 16)) * 16
        zbuf[r, pl.ds(c, 16)] = jnp.zeros((16,), jnp.float32)
        return 0
    lax.fori_loop(0, ZR * (D // 16), zfill, 0)

    for t in range(T):
        ebase = t * E + wid * EPW

        def zero(i, _):
            pltpu.sync_copy(zbuf, acc.at[pl.ds(sid * RPS + i * ZR, ZR)])
            return 0
        lax.fori_loop(0, RPS // ZR, zero, 0)

        for q in range(NSL):
            pltpu.sync_copy(src_hbm.at[pl.ds(ebase + q * CH, CH)], sidx[q])
            pltpu.sync_copy(dst_hbm.at[pl.ds(ebase + q * CH, CH)], didx[q])
        plsc.subcore_barrier()
        for q in range(NSL):
            pltpu.make_async_copy(hn_hbm.at[sidx[q]], rows[q], gsem[q]).start()

        def grp(g, _):
            for q in range(NSL):
                j = g * NSL + q

                @pl.when(j < NCH)
                def _():
                    pltpu.make_async_copy(
                        hn_hbm.at[sidx[q]], rows[q], gsem[q]).wait()
                    pltpu.sync_copy(rows[q], acc.at[didx[q]], add=True)

                    @pl.when(j + NSL < NCH)
                    def _():
                        off = ebase + (j + NSL) * CH
                        pltpu.make_async_copy(
                            src_hbm.at[pl.ds(off, CH)], sidx[q], ssem[q]).start()
                        pltpu.make_async_copy(
                            dst_hbm.at[pl.ds(off, CH)], didx[q], dsem[q]).start()

            for q in range(NSL):
                j2 = (g + 1) * NSL + q

                @pl.when(j2 < NCH)
                def _():
                    off = ebase + j2 * CH
                    pltpu.make_async_copy(
                        src_hbm.at[pl.ds(off, CH)], sidx[q], ssem[q]).wait()
                    pltpu.make_async_copy(
                        dst_hbm.at[pl.ds(off, CH)], didx[q], dsem[q]).wait()
                    pltpu.make_async_copy(
                        hn_hbm.at[sidx[q]], rows[q], gsem[q]).start()
            return 0
        lax.fori_loop(0, NGI, grp, 0)
        plsc.subcore_barrier()

        pltpu.sync_copy(acc.at[pl.ds(sid * RPS, RPS)],
                        out_hbm.at[pl.ds((t * NC + cid) * N_PAD + sid * RPS, RPS)])
        plsc.subcore_barrier()


def _m1_body(x_ref, w_ref, deg_ref, out_ref):
    dsum = deg_ref[0, 0] + deg_ref[0, 1]
    dinv = lax.rsqrt(dsum + 1.0)
    h = jnp.dot(x_ref[0], w_ref[...], preferred_element_type=jnp.float32)
    out_ref[...] = h * dinv


def _tc_m1(x_pad, W1, degcol):
    return pl.pallas_call(
        _m1_body,
        grid=(T, NB),
        in_specs=[
            pl.BlockSpec((1, BN, D), lambda t, n: (t, n, 0)),
            pl.BlockSpec((D, H), lambda t, n: (0, 0)),
            pl.BlockSpec((1, NC, BN, 1), lambda t, n: (t, 0, n, 0)),
        ],
        out_specs=pl.BlockSpec((BN, H), lambda t, n: (t * NB + n, 0)),
        out_shape=jax.ShapeDtypeStruct((T * N_PAD, H), jnp.float32),
    )(x_pad, W1, degcol)


def _m2_body(acc_ref, hn_ref, deg_ref, b_ref, w_ref, out_ref):
    dsum = deg_ref[0, 0] + deg_ref[0, 1]
    dinv = lax.rsqrt(dsum + 1.0)
    s = acc_ref[0, 0] + acc_ref[0, 1] + hn_ref[...]
    h1 = jnp.maximum(s * dinv + b_ref[...], 0.0)
    out_ref[...] = jnp.dot(h1, w_ref[...], preferred_element_type=jnp.float32) * dinv


def _tc_m2(acc1, hn1, degcol, b1r, W2):
    return pl.pallas_call(
        _m2_body,
        grid=(T, NB),
        in_specs=[
            pl.BlockSpec((1, NC, BN, D), lambda t, n: (t, 0, n, 0)),
            pl.BlockSpec((BN, D), lambda t, n: (t * NB + n, 0)),
            pl.BlockSpec((1, NC, BN, 1), lambda t, n: (t, 0, n, 0)),
            pl.BlockSpec((1, H), lambda t, n: (0, 0)),
            pl.BlockSpec((H, H), lambda t, n: (0, 0)),
        ],
        out_specs=pl.BlockSpec((BN, H), lambda t, n: (t * NB + n, 0)),
        out_shape=jax.ShapeDtypeStruct((T * N_PAD, H), jnp.float32),
    )(acc1, hn1, degcol, b1r, W2)


def _f_body(acc_ref, hn_ref, deg_ref, b_ref, out_ref):
    n = pl.program_id(1)
    dsum = deg_ref[0, 0] + deg_ref[0, 1]
    dinv = lax.rsqrt(dsum + 1.0)
    s = acc_ref[0, 0] + acc_ref[0, 1] + hn_ref[...]
    h2 = jnp.maximum(s * dinv + b_ref[...], 0.0)
    rowid = lax.broadcasted_iota(jnp.int32, (BN, 1), 0) + n * BN
    h2 = jnp.where(rowid < N, h2, 0.0)

    @pl.when(n == 0)
    def _():
        out_ref[...] = jnp.zeros_like(out_ref)

    out_ref[...] += jnp.sum(h2, axis=0, keepdims=True)[None]


def _tc_f(acc2, hn2, degcol, b2r):
    return pl.pallas_call(
        _f_body,
        grid=(T, NB),
        in_specs=[
            pl.BlockSpec((1, NC, BN, D), lambda t, n: (t, 0, n, 0)),
            pl.BlockSpec((BN, D), lambda t, n: (t * NB + n, 0)),
            pl.BlockSpec((1, NC, BN, 1), lambda t, n: (t, 0, n, 0)),
            pl.BlockSpec((1, H), lambda t, n: (0, 0)),
        ],
        out_specs=pl.BlockSpec((1, 1, H), lambda t, n: (t, 0, 0)),
        out_shape=jax.ShapeDtypeStruct((T, 1, H), jnp.float32),
    )(acc2, hn2, degcol, b2r)


def _gru_body(g_ref, wih_ref, whh_ref, bih_ref, bhh_ref, wh_ref, bh_ref, out_ref):
    g = g_ref[...] * (1.0 / N)
    wih = wih_ref[...]
    whh = whh_ref[...]
    bih = bih_ref[...]
    bhh = bhh_ref[...]
    dn = (((1,), (1,)), ((), ()))
    h = jnp.zeros((1, H), jnp.float32)
    for t in range(T):
        xt = g[t:t + 1, :]
        gi = lax.dot_general(xt, wih, dn, preferred_element_type=jnp.float32) + bih
        gh = lax.dot_general(h, whh, dn, preferred_element_type=jnp.float32) + bhh
        r = jax.nn.sigmoid(gi[:, :H] + gh[:, :H])
        z = jax.nn.sigmoid(gi[:, H:2 * H] + gh[:, H:2 * H])
        n_ = jnp.tanh(gi[:, 2 * H:] + r * gh[:, 2 * H:])
        h = (1.0 - z) * n_ + z * h
    out_ref[...] = lax.dot_general(h, wh_ref[...], dn,
                                   preferred_element_type=jnp.float32) + bh_ref[...]


def _tc_gru(g, W_ih, W_hh, b_ihr, b_hhr, Wh, bhr):
    return pl.pallas_call(
        _gru_body,
        out_shape=jax.ShapeDtypeStruct((1, D), jnp.float32),
    )(g, W_ih, W_hh, b_ihr, b_hhr, Wh, bhr)


def kernel(x_seq, ei_seq, W1, b1, W2, b2, W_ih, W_hh, b_ih, b_hh, Wh, bh):
    src = ei_seq[:, 0, :]
    srcf = (src + (jnp.arange(T, dtype=jnp.int32) * N_PAD)[:, None]).reshape(T * E)
    dstf = ei_seq[:, 1, :].reshape(T * E)
    x_pad = jnp.concatenate(
        [x_seq, jnp.zeros((T, N_PAD - N, D), jnp.float32)], axis=1)

    deg2 = _sc_deg(dstf)
    degcol = deg2.reshape(T, NC, N_PAD, 1)
    hn1 = _tc_m1(x_pad, W1, degcol)
    acc1 = _sc_edge(hn1, srcf, dstf).reshape(T, NC, N_PAD, H)
    hn2 = _tc_m2(acc1, hn1, degcol, b1.reshape(1, H), W2)
    acc2 = _sc_edge(hn2, srcf, dstf).reshape(T, NC, N_PAD, H)
    gsum = _tc_f(acc2, hn2, degcol, b2.reshape(1, H))
    out = _tc_gru(gsum.reshape(T, H), W_ih, W_hh,
                  b_ih.reshape(1, 3 * H), b_hh.reshape(1, 3 * H),
                  Wh, bh.reshape(1, D))
    return out.reshape(D)


# speedup vs baseline: 19.2165x; 1.0560x over previous
import functools

import jax
import jax.numpy as jnp
from jax import lax
from jax.experimental import pallas as pl
from jax.experimental.pallas import tpu as pltpu
from jax.experimental.pallas import tpu_sc as plsc

T = 8
N = 10000
N_PAD = 10240
E = 320000
D = 128
H = 128

NC = 2
NS = 16
NW = NC * NS
EPW = E // NW
CH = 80
NCH = EPW // CH
NSL = 3
NGI = (NCH + NSL - 1) // NSL
RPS = N_PAD // NS
ZR = 16

BN = 1024
NB = N_PAD // BN


def _mesh():
    return plsc.VectorSubcoreMesh(core_axis_name="c", subcore_axis_name="s")


DCH = 80
DNCH = EPW // DCH
DSL = 2
DNG = (DNCH + DSL - 1) // DSL


@functools.partial(
    pl.kernel,
    mesh=_mesh(),
    out_type=jax.ShapeDtypeStruct((T * NC * N_PAD,), jnp.float32),
    scratch_types=[
        pltpu.VMEM((DCH,), jnp.int32),
        pltpu.VMEM((DCH,), jnp.int32),
        pltpu.VMEM((DCH,), jnp.float32),
        pltpu.VMEM((RPS,), jnp.float32),
        pltpu.VMEM_SHARED((N_PAD,), jnp.float32),
        pltpu.SemaphoreType.DMA,
        pltpu.SemaphoreType.DMA,
    ],
)
def _sc_deg(dst_hbm, out_hbm, didx0, didx1, ones_v, zv, acc, sem0, sem1):
    didx = (didx0, didx1)
    dsem = (sem0, sem1)
    cid = lax.axis_index("c")
    sid = lax.axis_index("s")
    wid = cid * NS + sid

    def fill(i, _):
        ones_v[pl.ds(i * 16, 16)] = jnp.full((16,), 1.0, jnp.float32)
        return 0
    lax.fori_loop(0, DCH // 16, fill, 0)

    def zfill(i, _):
        zv[pl.ds(i * 16, 16)] = jnp.zeros((16,), jnp.float32)
        return 0
    lax.fori_loop(0, RPS // 16, zfill, 0)

    for t in range(T):
        ebase = t * E + wid * EPW
        pltpu.sync_copy(zv, acc.at[pl.ds(sid * RPS, RPS)])
        for q in range(DSL):
            pltpu.sync_copy(dst_hbm.at[pl.ds(ebase + q * DCH, DCH)], didx[q])
        plsc.subcore_barrier()

        def grp(g, _):
            for q in range(DSL):
                j = g * DSL + q

                @pl.when(j < DNCH)
                def _():
                    pltpu.sync_copy(ones_v, acc.at[didx[q]], add=True)

                    @pl.when(j + DSL < DNCH)
                    def _():
                        pltpu.make_async_copy(
                            dst_hbm.at[pl.ds(ebase + (j + DSL) * DCH, DCH)],
                            didx[q], dsem[q]).start()

            for q in range(DSL):
                j2 = (g + 1) * DSL + q

                @pl.when(j2 < DNCH)
                def _():
                    pltpu.make_async_copy(
                        dst_hbm.at[pl.ds(ebase + j2 * DCH, DCH)],
                        didx[q], dsem[q]).wait()
            return 0
        lax.fori_loop(0, DNG, grp, 0)
        plsc.subcore_barrier()

        pltpu.sync_copy(acc.at[pl.ds(sid * RPS, RPS)],
                        out_hbm.at[pl.ds((t * NC + cid) * N_PAD + sid * RPS, RPS)])
        plsc.subcore_barrier()


@functools.partial(
    pl.kernel,
    mesh=_mesh(),
    out_type=jax.ShapeDtypeStruct((T * NC * N_PAD, D), jnp.float32),
    scratch_types=(
        [pltpu.VMEM((CH,), jnp.int32) for _ in range(NSL)]
        + [pltpu.VMEM((CH,), jnp.int32) for _ in range(NSL)]
        + [pltpu.VMEM((CH, D), jnp.float32) for _ in range(NSL)]
        + [pltpu.VMEM((ZR, D), jnp.float32)]
        + [pltpu.VMEM_SHARED((N_PAD, D), jnp.float32)]
        + [pltpu.SemaphoreType.DMA] * (3 * NSL)
    ),
)
def _sc_edge(hn_hbm, src_hbm, dst_hbm, out_hbm, *refs):
    sidx = refs[0:NSL]
    didx = refs[NSL:2 * NSL]
    rows = refs[2 * NSL:3 * NSL]
    zbuf = refs[3 * NSL]
    acc = refs[3 * NSL + 1]
    ssem = refs[3 * NSL + 2:3 * NSL + 2 + NSL]
    dsem = refs[3 * NSL + 2 + NSL:3 * NSL + 2 + 2 * NSL]
    gsem = refs[3 * NSL + 2 + 2 * NSL:3 * NSL + 2 + 3 * NSL]

    cid = lax.axis_index("c")
    sid = lax.axis_index("s")
    wid = cid * NS + sid

    def zfill(i, _):
        r = i // (D // 16)
        c = (i % (D // 16)) * 16
        zbuf[r, pl.ds(c, 16)] = jnp.zeros((16,), jnp.float32)
        return 0
    lax.fori_loop(0, ZR * (D // 16), zfill, 0)

    for t in range(T):
        ebase = t * E + wid * EPW

        def zero(i, _):
            pltpu.sync_copy(zbuf, acc.at[pl.ds(sid * RPS + i * ZR, ZR)])
            return 0
        lax.fori_loop(0, RPS // ZR, zero, 0)

        for q in range(NSL):
            pltpu.sync_copy(src_hbm.at[pl.ds(ebase + q * CH, CH)], sidx[q])
            pltpu.sync_copy(dst_hbm.at[pl.ds(ebase + q * CH, CH)], didx[q])
        plsc.subcore_barrier()
        for q in range(NSL):
            pltpu.make_async_copy(hn_hbm.at[sidx[q]], rows[q], gsem[q]).start()

        def grp(g, _):
            for q in range(NSL):
                j = g * NSL + q

                @pl.when(j < NCH)
                def _():
                    pltpu.make_async_copy(
                        hn_hbm.at[sidx[q]], rows[q], gsem[q]).wait()
                    pltpu.sync_copy(rows[q], acc.at[didx[q]], add=True)

                    @pl.when(j + NSL < NCH)
                    def _():
                        off = ebase + (j + NSL) * CH
                        pltpu.make_async_copy(
                            src_hbm.at[pl.ds(off, CH)], sidx[q], ssem[q]).start()
                        pltpu.make_async_copy(
                            dst_hbm.at[pl.ds(off, CH)], didx[q], dsem[q]).start()

            for q in range(NSL):
                j2 = (g + 1) * NSL + q

                @pl.when(j2 < NCH)
                def _():
                    off = ebase + j2 * CH
                    pltpu.make_async_copy(
                        src_hbm.at[pl.ds(off, CH)], sidx[q], ssem[q]).wait()
                    pltpu.make_async_copy(
                        dst_hbm.at[pl.ds(off, CH)], didx[q], dsem[q]).wait()
                    pltpu.make_async_copy(
                        hn_hbm.at[sidx[q]], rows[q], gsem[q]).start()
            return 0
        lax.fori_loop(0, NGI, grp, 0)
        plsc.subcore_barrier()

        pltpu.sync_copy(acc.at[pl.ds(sid * RPS, RPS)],
                        out_hbm.at[pl.ds((t * NC + cid) * N_PAD + sid * RPS, RPS)])
        plsc.subcore_barrier()


def _m1_body(x_ref, w_ref, deg_ref, out_ref):
    dsum = deg_ref[0, 0] + deg_ref[0, 1]
    dinv = lax.rsqrt(dsum + 1.0)
    h = jnp.dot(x_ref[0], w_ref[...], preferred_element_type=jnp.float32)
    out_ref[...] = h * dinv


def _tc_m1(x_pad, W1, degcol):
    return pl.pallas_call(
        _m1_body,
        grid=(T, NB),
        in_specs=[
            pl.BlockSpec((1, BN, D), lambda t, n: (t, n, 0)),
            pl.BlockSpec((D, H), lambda t, n: (0, 0)),
            pl.BlockSpec((1, NC, BN, 1), lambda t, n: (t, 0, n, 0)),
        ],
        out_specs=pl.BlockSpec((BN, H), lambda t, n: (t * NB + n, 0)),
        out_shape=jax.ShapeDtypeStruct((T * N_PAD, H), jnp.float32),
    )(x_pad, W1, degcol)


def _m2_body(acc_ref, hn_ref, deg_ref, b_ref, w_ref, out_ref):
    dsum = deg_ref[0, 0] + deg_ref[0, 1]
    dinv = lax.rsqrt(dsum + 1.0)
    s = acc_ref[0, 0] + acc_ref[0, 1] + hn_ref[...]
    h1 = jnp.maximum(s * dinv + b_ref[...], 0.0)
    out_ref[...] = jnp.dot(h1, w_ref[...], preferred_element_type=jnp.float32) * dinv


def _tc_m2(acc1, hn1, degcol, b1r, W2):
    return pl.pallas_call(
        _m2_body,
        grid=(T, NB),
        in_specs=[
            pl.BlockSpec((1, NC, BN, D), lambda t, n: (t, 0, n, 0)),
            pl.BlockSpec((BN, D), lambda t, n: (t * NB + n, 0)),
            pl.BlockSpec((1, NC, BN, 1), lambda t, n: (t, 0, n, 0)),
            pl.BlockSpec((1, H), lambda t, n: (0, 0)),
            pl.BlockSpec((H, H), lambda t, n: (0, 0)),
        ],
        out_specs=pl.BlockSpec((BN, H), lambda t, n: (t * NB + n, 0)),
        out_shape=jax.ShapeDtypeStruct((T * N_PAD, H), jnp.float32),
    )(acc1, hn1, degcol, b1r, W2)


def _f_body(acc_ref, hn_ref, deg_ref, b_ref, out_ref):
    n = pl.program_id(1)
    dsum = deg_ref[0, 0] + deg_ref[0, 1]
    dinv = lax.rsqrt(dsum + 1.0)
    s = acc_ref[0, 0] + acc_ref[0, 1] + hn_ref[...]
    h2 = jnp.maximum(s * dinv + b_ref[...], 0.0)
    rowid = lax.broadcasted_iota(jnp.int32, (BN, 1), 0) + n * BN
    h2 = jnp.where(rowid < N, h2, 0.0)

    @pl.when(n == 0)
    def _():
        out_ref[...] = jnp.zeros_like(out_ref)

    out_ref[...] += jnp.sum(h2, axis=0, keepdims=True)[None]


def _tc_f(acc2, hn2, degcol, b2r):
    return pl.pallas_call(
        _f_body,
        grid=(T, NB),
        in_specs=[
            pl.BlockSpec((1, NC, BN, D), lambda t, n: (t, 0, n, 0)),
            pl.BlockSpec((BN, D), lambda t, n: (t * NB + n, 0)),
            pl.BlockSpec((1, NC, BN, 1), lambda t, n: (t, 0, n, 0)),
            pl.BlockSpec((1, H), lambda t, n: (0, 0)),
        ],
        out_specs=pl.BlockSpec((1, 1, H), lambda t, n: (t, 0, 0)),
        out_shape=jax.ShapeDtypeStruct((T, 1, H), jnp.float32),
    )(acc2, hn2, degcol, b2r)


def _gru_body(g_ref, wih_ref, whh_ref, bih_ref, bhh_ref, wh_ref, bh_ref, out_ref):
    g = g_ref[...] * (1.0 / N)
    wih = wih_ref[...]
    whh = whh_ref[...]
    bih = bih_ref[...]
    bhh = bhh_ref[...]
    dn = (((1,), (1,)), ((), ()))
    h = jnp.zeros((1, H), jnp.float32)
    for t in range(T):
        xt = g[t:t + 1, :]
        gi = lax.dot_general(xt, wih, dn, preferred_element_type=jnp.float32) + bih
        gh = lax.dot_general(h, whh, dn, preferred_element_type=jnp.float32) + bhh
        r = jax.nn.sigmoid(gi[:, :H] + gh[:, :H])
        z = jax.nn.sigmoid(gi[:, H:2 * H] + gh[:, H:2 * H])
        n_ = jnp.tanh(gi[:, 2 * H:] + r * gh[:, 2 * H:])
        h = (1.0 - z) * n_ + z * h
    out_ref[...] = lax.dot_general(h, wh_ref[...], dn,
                                   preferred_element_type=jnp.float32) + bh_ref[...]


def _tc_gru(g, W_ih, W_hh, b_ihr, b_hhr, Wh, bhr):
    return pl.pallas_call(
        _gru_body,
        out_shape=jax.ShapeDtypeStruct((1, D), jnp.float32),
    )(g, W_ih, W_hh, b_ihr, b_hhr, Wh, bhr)


def kernel(x_seq, ei_seq, W1, b1, W2, b2, W_ih, W_hh, b_ih, b_hh, Wh, bh):
    src = ei_seq[:, 0, :]
    srcf = (src + (jnp.arange(T, dtype=jnp.int32) * N_PAD)[:, None]).reshape(T * E)
    dstf = ei_seq[:, 1, :].reshape(T * E)
    x_pad = jnp.concatenate(
        [x_seq, jnp.zeros((T, N_PAD - N, D), jnp.float32)], axis=1)

    deg2 = _sc_deg(dstf)
    degcol = deg2.reshape(T, NC, N_PAD, 1)
    hn1 = _tc_m1(x_pad, W1, degcol)
    acc1 = _sc_edge(hn1, srcf, dstf).reshape(T, NC, N_PAD, H)
    hn2 = _tc_m2(acc1, hn1, degcol, b1.reshape(1, H), W2)
    acc2 = _sc_edge(hn2, srcf, dstf).reshape(T, NC, N_PAD, H)
    gsum = _tc_f(acc2, hn2, degcol, b2.reshape(1, H))
    out = _tc_gru(gsum.reshape(T, H), W_ih, W_hh,
                  b_ih.reshape(1, 3 * H), b_hh.reshape(1, 3 * H),
                  Wh, bh.reshape(1, D))
    return out.reshape(D)
